# baseline (speedup 1.0000x reference)
import functools

import jax
import jax.numpy as jnp
from jax import lax
from jax.experimental import pallas as pl
from jax.experimental.pallas import tpu as pltpu
from jax.experimental.pallas import tpu_sc as plsc

B, N, D = 32, 8192, 64
C = 129
NC, NS = 2, 16

TOK = N // NS
CH1 = 256
NCHK1 = TOK // CH1
SUB1 = CH1 // 128

CH3 = 512
NCH3 = N // CH3
SUB3 = CH3 // 128

_mesh = plsc.VectorSubcoreMesh(
    core_axis_name="c", subcore_axis_name="s", num_cores=NC, num_subcores=NS)
_sc_params = pltpu.CompilerParams(use_tc_tiling_on_sc=False)


@functools.partial(
    pl.kernel,
    out_type=(
        jax.ShapeDtypeStruct((B, C, D), jnp.float32),
        jax.ShapeDtypeStruct((B, C, D), jnp.float32),
        jax.ShapeDtypeStruct((B, C, D), jnp.float32),
        jax.ShapeDtypeStruct((B, C, D), jnp.float32),
    ),
    mesh=_mesh,
    scratch_types=[
        pltpu.VMEM((SUB1 * NCHK1, 128), jnp.int32),
        pltpu.VMEM((CH1, D), jnp.float32),
        pltpu.VMEM((CH1, D), jnp.float32),
        pltpu.VMEM((CH1, D), jnp.float32),
        pltpu.VMEM((128, D), jnp.float32),
        pltpu.VMEM_SHARED((C, D), jnp.float32),
        pltpu.VMEM_SHARED((C, D), jnp.float32),
        pltpu.VMEM_SHARED((C, D), jnp.float32),
        pltpu.VMEM_SHARED((C, D), jnp.float32),
    ],
    compiler_params=_sc_params,
)
def _seg_sums(q_hbm, k_hbm, v_hbm, cl_hbm, zeros_hbm, ones_hbm,
              segq_hbm, segk_hbm, segv_hbm, cnt_hbm,
              idx_v, qb, kb, vb, ones_v, aq, ak, av, ac):
    s = lax.axis_index("s")
    c = lax.axis_index("c")
    pltpu.sync_copy(ones_hbm, ones_v)

    def batch_body(i, carry):
        b = i * NC + c

        @pl.when(s == 0)
        def _zero():
            pltpu.sync_copy(zeros_hbm, aq)
            pltpu.sync_copy(zeros_hbm, ak)
            pltpu.sync_copy(zeros_hbm, av)
            pltpu.sync_copy(zeros_hbm, ac)

        plsc.subcore_barrier()

        pltpu.sync_copy(cl_hbm.at[b, pl.ds(s * (TOK // 128), TOK // 128)],
                        idx_v)

        def chunk_body(t, carry2):
            tok0 = s * TOK + t * CH1
            pltpu.sync_copy(q_hbm.at[b, pl.ds(tok0, CH1)], qb)
            pltpu.sync_copy(k_hbm.at[b, pl.ds(tok0, CH1)], kb)
            pltpu.sync_copy(v_hbm.at[b, pl.ds(tok0, CH1)], vb)
            for j in range(SUB1):
                row = idx_v.at[t * SUB1 + j]
                sl = pl.ds(j * 128, 128)
                pltpu.sync_copy(qb.at[sl], aq.at[row], add=True)
                pltpu.sync_copy(kb.at[sl], ak.at[row], add=True)
                pltpu.sync_copy(vb.at[sl], av.at[row], add=True)
                pltpu.sync_copy(ones_v, ac.at[row], add=True)
            return carry2

        lax.fori_loop(0, NCHK1, chunk_body, 0)

        plsc.subcore_barrier()

        @pl.when(s == 0)
        def _writeout():
            pltpu.sync_copy(aq, segq_hbm.at[b])
            pltpu.sync_copy(ak, segk_hbm.at[b])
            pltpu.sync_copy(av, segv_hbm.at[b])
            pltpu.sync_copy(ac, cnt_hbm.at[b])

        return carry

    lax.fori_loop(0, NS, batch_body, 0)


def _attn_body(segq_ref, segk_ref, segv_ref, cnt_ref, v2_ref, acol_ref):
    col0 = (lax.broadcasted_iota(jnp.int32, (C, C), 1) == 0).astype(jnp.float32)
    for b in range(B):
        cnt = jnp.sum(cnt_ref[b], axis=1) * (1.0 / D)
        inv = 1.0 / cnt
        qc = segq_ref[b] * inv[:, None]
        kc = segk_ref[b] * inv[:, None]
        vc = segv_ref[b] * inv[:, None]

        qk = lax.dot_general(qc, kc, (((1,), (1,)), ((), ())),
                             preferred_element_type=jnp.float32)
        a = jax.nn.softmax(qk, axis=-1)
        aw = a * cnt[None, :]
        aw = aw / jnp.sum(aw, axis=-1, keepdims=True)

        v2_ref[b] = jnp.dot(aw, vc, preferred_element_type=jnp.float32)
        acol_ref[b, 0, :] = jnp.sum(aw * col0, axis=1)


def _attn(segq, segk, segv, cntd):
    return pl.pallas_call(
        _attn_body,
        out_shape=[
            jax.ShapeDtypeStruct((B, C, D), jnp.float32),
            jax.ShapeDtypeStruct((B, 1, C), jnp.float32),
        ],
    )(segq, segk, segv, cntd)


@functools.partial(
    pl.kernel,
    out_type=jax.ShapeDtypeStruct((B, N, D), jnp.float32),
    mesh=_mesh,
    scratch_types=[
        pltpu.VMEM((SUB3, 128), jnp.int32),
        pltpu.VMEM((CH3, D), jnp.float32),
        pltpu.SemaphoreType.DMA,
    ],
    compiler_params=_sc_params,
)
def _bcast_gather(v2_hbm, gcl_hbm, out_hbm, idx_v, rows, sem):
    b = lax.axis_index("s") * NC + lax.axis_index("c")

    def body(ci, carry):
        pltpu.sync_copy(gcl_hbm.at[b, pl.ds(ci * SUB3, SUB3)], idx_v)
        for j in range(SUB3):
            pltpu.async_copy(v2_hbm.at[idx_v.at[j]],
                             rows.at[pl.ds(j * 128, 128)], sem).wait()
        pltpu.sync_copy(rows, out_hbm.at[b, pl.ds(ci * CH3, CH3)])
        return carry

    lax.fori_loop(0, NCH3, body, 0)


def kernel(queries, keys, values, clusters):
    zeros = jnp.zeros((C, D), jnp.float32)
    ones = jnp.ones((128, D), jnp.float32)
    cl3 = clusters.reshape(B, N // 128, 128)

    segq, segk, segv, cntd = _seg_sums(queries, keys, values, cl3, zeros, ones)
    v2, acol = _attn(segq, segk, segv, cntd)

    gcl = (clusters + C * jnp.arange(B, dtype=jnp.int32)[:, None])
    gcl = gcl.reshape(B, N // 128, 128)
    out = _bcast_gather(v2.reshape(B * C, D), gcl)
    return (out, acol.reshape(B, C))
